# baseline (speedup 1.0000x reference)
import functools

import jax
import jax.numpy as jnp
from jax import lax
from jax.experimental import pallas as pl
from jax.experimental.pallas import tpu as pltpu
from jax.experimental.pallas import tpu_sc as plsc

BATCH = 1024
SEQ = 50
SEQ_PAD = 56
DIM = 64
VOCAB = 1000
VOCAB_PAD = 1024
NUM_WORKERS = 32
D_BLOCKS = DIM // 8
B_BLOCKS = BATCH // 128

_mesh = plsc.VectorSubcoreMesh(core_axis_name="c", subcore_axis_name="s")


@functools.partial(
    pl.kernel,
    mesh=_mesh,
    out_type=jax.ShapeDtypeStruct((SEQ, DIM, BATCH), jnp.float32),
    scratch_types=[
        pltpu.VMEM((DIM, VOCAB), jnp.float32),
        pltpu.VMEM((SEQ, 256), jnp.int32),
        pltpu.VMEM((2, 8, 128), jnp.float32),
        pltpu.SemaphoreType.DMA,
        pltpu.SemaphoreType.DMA,
        pltpu.SemaphoreType.DMA,
    ],
    compiler_params=pltpu.CompilerParams(
        use_tc_tiling_on_sc=True, needs_layout_passes=False
    ),
)
def _emb_lookup(ids_hbm, table_hbm, out_hbm, tab_v, ids_v, tile_v, sem, os0, os1):
    wid = lax.axis_index("s") * 2 + lax.axis_index("c")
    unit0 = wid * 2
    dblk = unit0 // B_BLOCKS
    bblk0 = unit0 % B_BLOCKS
    osem = (os0, os1)
    tcopy = pltpu.async_copy(table_hbm, tab_v, sem)
    pltpu.sync_copy(ids_hbm.at[:, pl.ds(bblk0 * 128, 256)], ids_v)
    tcopy.wait()

    def make_tile(u, s, buf):
        for v in range(8):
            idx16 = ids_v[s, pl.ds(u * 128 + v * 16, 16)]
            gathered = [
                plsc.load_gather(
                    tab_v,
                    [jnp.full((16,), dblk * 8 + d8, jnp.int32), idx16],
                )
                for d8 in range(8)
            ]
            for d8 in range(8):
                tile_v[buf, d8, pl.ds(v * 16, 16)] = gathered[d8]

    def dst(u, s):
        return out_hbm.at[s, pl.ds(dblk * 8, 8), pl.ds((bblk0 + u) * 128, 128)]

    for u in range(2):
        for b in range(2):
            make_tile(u, b, b)
            pltpu.async_copy(tile_v.at[b], dst(u, b), osem[b])

        @pl.loop(2, SEQ, step=2)
        def seq_body(s0):
            for b in range(2):
                s = s0 + b
                pltpu.make_async_copy(tile_v.at[b], dst(u, s - 2), osem[b]).wait()
                make_tile(u, s, b)
                pltpu.async_copy(tile_v.at[b], dst(u, s), osem[b])

        for b in range(2):
            pltpu.make_async_copy(tile_v.at[b], dst(u, SEQ - 2 + b), osem[b]).wait()


def kernel(token_ids, embedding_lookup):
    out = _emb_lookup(token_ids.astype(jnp.int32).T, embedding_lookup.T)
    return jnp.transpose(out, (2, 0, 1))


# speedup vs baseline: 1.2999x; 1.2999x over previous
import functools

import jax
import jax.numpy as jnp
from jax import lax
from jax.experimental import pallas as pl
from jax.experimental.pallas import tpu as pltpu
from jax.experimental.pallas import tpu_sc as plsc

BATCH = 1024
SEQ = 50
SEQ_PAD = 56
DIM = 64
VOCAB = 1000
VOCAB_PAD = 1024
NUM_WORKERS = 32
D_BLOCKS = DIM // 8
B_BLOCKS = BATCH // 128

_mesh = plsc.VectorSubcoreMesh(core_axis_name="c", subcore_axis_name="s")


@functools.partial(
    pl.kernel,
    mesh=_mesh,
    out_type=jax.ShapeDtypeStruct((SEQ, DIM, BATCH), jnp.float32),
    scratch_types=[
        pltpu.VMEM((8, VOCAB), jnp.float32),
        pltpu.VMEM((SEQ, 256), jnp.int32),
        pltpu.VMEM((2, 8, 128), jnp.float32),
        pltpu.SemaphoreType.DMA,
        pltpu.SemaphoreType.DMA,
        pltpu.SemaphoreType.DMA,
    ],
    compiler_params=pltpu.CompilerParams(
        use_tc_tiling_on_sc=True, needs_layout_passes=False
    ),
)
def _emb_lookup(ids_hbm, table_hbm, out_hbm, tab_v, ids_v, tile_v, sem, os0, os1):
    wid = lax.axis_index("s") * 2 + lax.axis_index("c")
    unit0 = wid * 2
    dblk = unit0 // B_BLOCKS
    bblk0 = unit0 % B_BLOCKS
    osem = (os0, os1)
    tcopy = pltpu.async_copy(table_hbm.at[pl.ds(dblk * 8, 8)], tab_v, sem)
    pltpu.sync_copy(ids_hbm.at[:, pl.ds(bblk0 * 128, 256)], ids_v)
    tcopy.wait()

    def make_tile(u, s, buf):
        for vp in range(4):
            idx_pair = [
                ids_v[s, pl.ds(u * 128 + (2 * vp + h) * 16, 16)] for h in range(2)
            ]
            gathered = [
                [
                    plsc.load_gather(
                        tab_v,
                        [jnp.full((16,), d8, jnp.int32), idx_pair[h]],
                    )
                    for d8 in range(8)
                ]
                for h in range(2)
            ]
            for h in range(2):
                for d8 in range(8):
                    tile_v[buf, d8, pl.ds((2 * vp + h) * 16, 16)] = gathered[h][d8]

    def dst(u, s):
        return out_hbm.at[s, pl.ds(dblk * 8, 8), pl.ds((bblk0 + u) * 128, 128)]

    for u in range(2):
        for b in range(2):
            make_tile(u, b, b)
            pltpu.async_copy(tile_v.at[b], dst(u, b), osem[b])

        @pl.loop(2, SEQ, step=2)
        def seq_body(s0):
            for b in range(2):
                s = s0 + b
                pltpu.make_async_copy(tile_v.at[b], dst(u, s - 2), osem[b]).wait()
                make_tile(u, s, b)
                pltpu.async_copy(tile_v.at[b], dst(u, s), osem[b])

        for b in range(2):
            pltpu.make_async_copy(tile_v.at[b], dst(u, SEQ - 2 + b), osem[b]).wait()


def kernel(token_ids, embedding_lookup):
    out = _emb_lookup(token_ids.astype(jnp.int32).T, embedding_lookup.T)
    return jnp.transpose(out, (2, 0, 1))


# speedup vs baseline: 1.3116x; 1.0090x over previous
import functools

import jax
import jax.numpy as jnp
from jax import lax
from jax.experimental import pallas as pl
from jax.experimental.pallas import tpu as pltpu
from jax.experimental.pallas import tpu_sc as plsc

BATCH = 1024
SEQ = 50
SEQ_PAD = 56
DIM = 64
VOCAB = 1000
VOCAB_PAD = 1024
NUM_WORKERS = 32
D_BLOCKS = DIM // 8
B_BLOCKS = BATCH // 128

_mesh = plsc.VectorSubcoreMesh(core_axis_name="c", subcore_axis_name="s")


@functools.partial(
    pl.kernel,
    mesh=_mesh,
    out_type=jax.ShapeDtypeStruct((SEQ, DIM, BATCH), jnp.float32),
    scratch_types=[
        pltpu.VMEM((8, VOCAB), jnp.float32),
        pltpu.VMEM((SEQ, 256), jnp.int32),
        pltpu.VMEM((2, 8, 256), jnp.float32),
        pltpu.SemaphoreType.DMA,
        pltpu.SemaphoreType.DMA,
        pltpu.SemaphoreType.DMA,
    ],
    compiler_params=pltpu.CompilerParams(
        use_tc_tiling_on_sc=True, needs_layout_passes=False
    ),
)
def _emb_lookup(ids_hbm, table_hbm, out_hbm, tab_v, ids_v, tile_v, sem, os0, os1):
    wid = lax.axis_index("s") * 2 + lax.axis_index("c")
    unit0 = wid * 2
    dblk = unit0 // B_BLOCKS
    bblk0 = unit0 % B_BLOCKS
    osem = (os0, os1)
    tcopy = pltpu.async_copy(table_hbm.at[pl.ds(dblk * 8, 8)], tab_v, sem)
    pltpu.sync_copy(ids_hbm.at[:, pl.ds(bblk0 * 128, 256)], ids_v)
    tcopy.wait()

    def make_tiles(s, buf):
        for v2 in range(8):
            idx_pair = [
                ids_v[s, pl.ds((2 * v2 + h) * 16, 16)] for h in range(2)
            ]
            gathered = [
                [
                    plsc.load_gather(
                        tab_v,
                        [jnp.full((16,), d8, jnp.int32), idx_pair[h]],
                    )
                    for d8 in range(8)
                ]
                for h in range(2)
            ]
            for h in range(2):
                for d8 in range(8):
                    tile_v[buf, d8, pl.ds((2 * v2 + h) * 16, 16)] = gathered[h][d8]

    def dst(s):
        return out_hbm.at[s, pl.ds(dblk * 8, 8), pl.ds(bblk0 * 128, 256)]

    for b in range(2):
        make_tiles(b, b)
        pltpu.async_copy(tile_v.at[b], dst(b), osem[b])

    @pl.loop(2, SEQ, step=2)
    def seq_body(s0):
        for b in range(2):
            s = s0 + b
            pltpu.make_async_copy(tile_v.at[b], dst(s - 2), osem[b]).wait()
            make_tiles(s, b)
            pltpu.async_copy(tile_v.at[b], dst(s), osem[b])

    for b in range(2):
        pltpu.make_async_copy(tile_v.at[b], dst(SEQ - 2 + b), osem[b]).wait()


def kernel(token_ids, embedding_lookup):
    out = _emb_lookup(token_ids.astype(jnp.int32).T, embedding_lookup.T)
    return jnp.transpose(out, (2, 0, 1))


# speedup vs baseline: 1.3132x; 1.0012x over previous
import functools

import jax
import jax.numpy as jnp
from jax import lax
from jax.experimental import pallas as pl
from jax.experimental.pallas import tpu as pltpu
from jax.experimental.pallas import tpu_sc as plsc

BATCH = 1024
SEQ = 50
SEQ_PAD = 56
DIM = 64
VOCAB = 1000
VOCAB_PAD = 1024
NUM_WORKERS = 32
D_BLOCKS = DIM // 8
B_BLOCKS = BATCH // 128

_mesh = plsc.VectorSubcoreMesh(core_axis_name="c", subcore_axis_name="s")


@functools.partial(
    pl.kernel,
    mesh=_mesh,
    out_type=jax.ShapeDtypeStruct((SEQ, DIM, BATCH), jnp.float32),
    scratch_types=[
        pltpu.VMEM((8, VOCAB), jnp.float32),
        pltpu.VMEM((SEQ, 256), jnp.int32),
        pltpu.VMEM((2, 8, 256), jnp.float32),
        pltpu.SemaphoreType.DMA,
        pltpu.SemaphoreType.DMA,
        pltpu.SemaphoreType.DMA,
    ],
    compiler_params=pltpu.CompilerParams(
        use_tc_tiling_on_sc=True, needs_layout_passes=False, skip_device_barrier=True
    ),
)
def _emb_lookup(ids_hbm, table_hbm, out_hbm, tab_v, ids_v, tile_v, sem, os0, os1):
    wid = lax.axis_index("s") * 2 + lax.axis_index("c")
    unit0 = wid * 2
    dblk = unit0 // B_BLOCKS
    bblk0 = unit0 % B_BLOCKS
    osem = (os0, os1)
    tcopy = pltpu.async_copy(table_hbm.at[pl.ds(dblk * 8, 8)], tab_v, sem)
    pltpu.sync_copy(ids_hbm.at[:, pl.ds(bblk0 * 128, 256)], ids_v)
    tcopy.wait()

    def make_tiles(s, buf):
        for v2 in range(8):
            idx_pair = [
                ids_v[s, pl.ds((2 * v2 + h) * 16, 16)] for h in range(2)
            ]
            gathered = [
                [
                    plsc.load_gather(
                        tab_v,
                        [jnp.full((16,), d8, jnp.int32), idx_pair[h]],
                    )
                    for d8 in range(8)
                ]
                for h in range(2)
            ]
            for h in range(2):
                for d8 in range(8):
                    tile_v[buf, d8, pl.ds((2 * v2 + h) * 16, 16)] = gathered[h][d8]

    def dst(s):
        return out_hbm.at[s, pl.ds(dblk * 8, 8), pl.ds(bblk0 * 128, 256)]

    for b in range(2):
        make_tiles(b, b)
        pltpu.async_copy(tile_v.at[b], dst(b), osem[b])

    @pl.loop(2, SEQ, step=2)
    def seq_body(s0):
        for b in range(2):
            s = s0 + b
            pltpu.make_async_copy(tile_v.at[b], dst(s - 2), osem[b]).wait()
            make_tiles(s, b)
            pltpu.async_copy(tile_v.at[b], dst(s), osem[b])

    for b in range(2):
        pltpu.make_async_copy(tile_v.at[b], dst(SEQ - 2 + b), osem[b]).wait()


def kernel(token_ids, embedding_lookup):
    out = _emb_lookup(token_ids.astype(jnp.int32).T, embedding_lookup.T)
    return jnp.transpose(out, (2, 0, 1))


# speedup vs baseline: 1.4070x; 1.0715x over previous
import functools

import jax
import jax.numpy as jnp
from jax import lax
from jax.experimental import pallas as pl
from jax.experimental.pallas import tpu as pltpu
from jax.experimental.pallas import tpu_sc as plsc

BATCH = 1024
SEQ = 50
SEQ_PAD = 56
DIM = 64
VOCAB = 1000
VOCAB_PAD = 1024
NUM_WORKERS = 32
D_BLOCKS = DIM // 8
B_BLOCKS = BATCH // 128

_mesh = plsc.VectorSubcoreMesh(core_axis_name="c", subcore_axis_name="s")


@functools.partial(
    pl.kernel,
    mesh=_mesh,
    out_type=jax.ShapeDtypeStruct((SEQ, DIM, BATCH), jnp.float32),
    scratch_types=[
        pltpu.VMEM((8, VOCAB), jnp.float32),
        pltpu.VMEM((SEQ, 256), jnp.int32),
        pltpu.VMEM((2, 8, 256), jnp.float32),
        pltpu.SemaphoreType.DMA,
        pltpu.SemaphoreType.DMA,
        pltpu.SemaphoreType.DMA,
    ],
    compiler_params=pltpu.CompilerParams(
        use_tc_tiling_on_sc=True, needs_layout_passes=False
    ),
)
def _emb_lookup(ids_hbm, table_hbm, out_hbm, tab_v, ids_v, tile_v, sem, os0, os1):
    wid = lax.axis_index("s") * 2 + lax.axis_index("c")
    unit0 = wid * 2
    dblk = unit0 // B_BLOCKS
    bblk0 = unit0 % B_BLOCKS
    osem = (os0, os1)
    tcopy = pltpu.async_copy(table_hbm.at[pl.ds(dblk * 8, 8)], tab_v, sem)
    pltpu.sync_copy(ids_hbm.at[:, pl.ds(bblk0 * 128, 256)], ids_v)
    tcopy.wait()

    def make_tiles(s, buf):
        idx = [ids_v[s, pl.ds(v * 16, 16)] for v in range(16)]
        for v2 in range(8):
            gathered = [
                [
                    plsc.load_gather(
                        tab_v,
                        [jnp.full((16,), d8, jnp.int32), idx[2 * v2 + h]],
                    )
                    for d8 in range(8)
                ]
                for h in range(2)
            ]
            for h in range(2):
                for d8 in range(8):
                    tile_v[buf, d8, pl.ds((2 * v2 + h) * 16, 16)] = gathered[h][d8]

    def dst(s):
        return out_hbm.at[s, pl.ds(dblk * 8, 8), pl.ds(bblk0 * 128, 256)]

    for b in range(2):
        make_tiles(b, b)
        pltpu.async_copy(tile_v.at[b], dst(b), osem[b])

    @pl.loop(2, SEQ, step=2)
    def seq_body(s0):
        for b in range(2):
            s = s0 + b
            pltpu.make_async_copy(tile_v.at[b], dst(s - 2), osem[b]).wait()
            make_tiles(s, b)
            pltpu.async_copy(tile_v.at[b], dst(s), osem[b])

    for b in range(2):
        pltpu.make_async_copy(tile_v.at[b], dst(SEQ - 2 + b), osem[b]).wait()


def kernel(token_ids, embedding_lookup):
    out = _emb_lookup(token_ids.astype(jnp.int32).T, embedding_lookup.T)
    return jnp.transpose(out, (2, 0, 1))
